# baseline (speedup 1.0000x reference)
import functools
import jax
import jax.numpy as jnp
from jax import lax
from jax.experimental import pallas as pl
from jax.experimental.pallas import tpu as pltpu
from jax.experimental.pallas import tpu_sc as plsc

NN = 10000
NP = 10240
EE = 320000
EP = 322560
CC = 128
CH = CC // 2
NSC = 2
NTEC = 16
CHUNK = 80
CPT = EP // NTEC // CHUNK
RPT = NP // NTEC
DEGW = 16

_MESH = dict(core_axis_name="c", subcore_axis_name="s", num_cores=NSC,
             num_subcores=NTEC)


def _sc_agg_body(do_deg, row_off, tbl, esrc, edst, *refs):
    if do_deg:
        (acc_out, deg_out, idx_src, idx_dst, rows, ones_v, zer_v,
         acc_sh, deg_sh, sem, dsem) = refs
    else:
        acc_out, idx_src, idx_dst, rows, acc_sh, sem = refs

    c = lax.axis_index("c")
    s = lax.axis_index("s")
    row0 = s * RPT

    pltpu.sync_copy(esrc.at[s], idx_src)
    pltpu.sync_copy(edst.at[s], idx_dst)

    pltpu.sync_copy(tbl.at[c, pl.ds(row_off + row0, RPT)],
                    acc_sh.at[pl.ds(row0, RPT)])

    if do_deg:
        for r in range(CHUNK):
            ones_v[r, :] = jnp.full((16,), 1.0, jnp.float32)

        @pl.when(c == 1)
        def _():
            for r in range(128):
                zer_v[r, :] = jnp.zeros((16,), jnp.float32)
            for b in range(RPT // 128):
                pltpu.sync_copy(zer_v, deg_sh.at[pl.ds(row0 + b * 128, 128)])

    plsc.subcore_barrier()

    gsem = sem

    def start_gather(j, i):
        pltpu.make_async_copy(tbl.at[c].at[idx_src.at[j]], rows[i],
                              gsem[i]).start()

    def finish_chunk(j, i):
        pltpu.make_async_copy(tbl.at[c].at[idx_src.at[j]], rows[i],
                              gsem[i]).wait()
        pltpu.sync_copy(rows[i], acc_sh.at[idx_dst.at[j]], add=True)
        if do_deg:
            @pl.when(c == 1)
            def _():
                pltpu.async_copy(ones_v, deg_sh.at[idx_dst.at[j]],
                                 dsem, add=True)

    start_gather(0, 0)
    start_gather(1, 1)

    def body(k, carry):
        for i in range(3):
            jj = 3 * k + i
            nxt = (i + 2) % 3

            @pl.when(jj + 2 < CPT)
            def _():
                start_gather(jj + 2, nxt)
            finish_chunk(jj, i)
        return carry

    lax.fori_loop(0, CPT // 3, body, 0, unroll=False)
    if do_deg:
        @pl.when(c == 1)
        def _():
            def drain(j, carry):
                pltpu.make_async_copy(ones_v, deg_sh.at[idx_dst.at[0]],
                                      dsem).wait()
                return carry
            lax.fori_loop(0, CPT, drain, 0, unroll=False)

    plsc.subcore_barrier()

    pltpu.sync_copy(acc_sh.at[pl.ds(row0, RPT)],
                    acc_out.at[c, pl.ds(row0, RPT)])

    if do_deg:
        @pl.when(c == 1)
        def _():
            pltpu.sync_copy(deg_sh.at[pl.ds(row0, RPT)],
                            deg_out.at[pl.ds(row0, RPT)])


def _sc_agg(tbl, esrc, edst, do_deg, row_off):
    out_acc = jax.ShapeDtypeStruct((NSC, NP, CH), jnp.float32)
    sems3 = tuple(pltpu.SemaphoreType.DMA for _ in range(3))
    scratch = [
        pltpu.VMEM((CPT, CHUNK), jnp.int32),
        pltpu.VMEM((CPT, CHUNK), jnp.int32),
        tuple(pltpu.VMEM((CHUNK, CH), jnp.float32) for _ in range(3)),
    ]
    if do_deg:
        out_type = (out_acc, jax.ShapeDtypeStruct((NP, DEGW), jnp.float32))
        scratch += [pltpu.VMEM((CHUNK, DEGW), jnp.float32),
                    pltpu.VMEM((128, DEGW), jnp.float32),
                    pltpu.VMEM_SHARED((NP, CH), jnp.float32),
                    pltpu.VMEM_SHARED((NP, DEGW), jnp.float32),
                    sems3, pltpu.SemaphoreType.DMA]
    else:
        out_type = out_acc
        scratch += [pltpu.VMEM_SHARED((NP, CH), jnp.float32),
                    sems3]
    return pl.kernel(
        functools.partial(_sc_agg_body, do_deg, row_off),
        out_type=out_type,
        mesh=plsc.VectorSubcoreMesh(**_MESH),
        scratch_types=scratch,
        compiler_params=pltpu.CompilerParams(use_tc_tiling_on_sc=False),
    )(tbl, esrc, edst)


def _tc_pre_body(s0, s1, wpre, gamma, beta, tbl_out):
    for i, sref in enumerate((s0, s1)):
        h = jnp.dot(sref[...], wpre[i], preferred_element_type=jnp.float32)
        mu = jnp.mean(h, axis=0, keepdims=True)
        var = jnp.mean(jnp.square(h - mu), axis=0, keepdims=True)
        hn = (h - mu) * lax.rsqrt(var + 1e-5) * gamma[i][None, :] + beta[i][None, :]
        p = jnp.maximum(hn, 0.0)
        for half in range(2):
            tbl_out[half, i * NP:i * NP + NN, :] = p[:, half * CH:(half + 1) * CH]


def _tc_pre(s0, s1, wpre, gamma, beta):
    return pl.pallas_call(
        _tc_pre_body,
        out_shape=jax.ShapeDtypeStruct((2, 2 * NP, CH), jnp.float32),
    )(s0, s1, wpre, gamma, beta)


BR = 1000
NB = NN // BR


def _unsplit_blk(a):
    return jnp.concatenate([a[0], a[1]], axis=1)


def _tc_mid_body(tblp0, tblp1, acc0, acc1, deg, wsage, wgcn0, s2o, s3o, sg2o):
    inv = 1.0 / (deg[:, 0:1] + 1.0)
    p0 = jnp.concatenate([tblp0[0, 0], tblp0[1, 0]], axis=1)
    p1 = jnp.concatenate([tblp1[0, 0], tblp1[1, 0]], axis=1)
    a0 = _unsplit_blk(acc0) * inv
    a1 = _unsplit_blk(acc1) * inv
    dot = lambda a, b: jnp.dot(a, b, preferred_element_type=jnp.float32)
    r = lambda x: jnp.maximum(x, 0.0)
    s2 = r(dot(p0, wsage[0, 0]) + dot(a0, wsage[0, 1])) + r(dot(a1, wgcn0[...]))
    for half in range(2):
        s2o[half] = s2[:, half * CH:(half + 1) * CH]
    s3o[...] = r(dot(p1, wsage[1, 0]) + dot(a1, wsage[1, 1])) + p0
    sg2o[...] = r(dot(p1, wsage[2, 0]) + dot(a1, wsage[2, 1]))


def _tc_mid(tbl4, acc0, acc1, deg, wsage, wgcn0):
    full = lambda *shape: pl.BlockSpec(shape, lambda i: (0,) * len(shape))
    return pl.pallas_call(
        _tc_mid_body,
        grid=(NB,),
        in_specs=[
            pl.BlockSpec((2, 1, BR, CH), lambda i: (0, 0, i, 0)),
            pl.BlockSpec((2, 1, BR, CH), lambda i: (0, 1, i, 0)),
            pl.BlockSpec((2, BR, CH), lambda i: (0, i, 0)),
            pl.BlockSpec((2, BR, CH), lambda i: (0, i, 0)),
            pl.BlockSpec((BR, DEGW), lambda i: (i, 0)),
            full(3, 2, CC, CC),
            full(CC, CC),
        ],
        out_specs=(
            pl.BlockSpec((2, BR, CH), lambda i: (0, i, 0)),
            pl.BlockSpec((BR, CC), lambda i: (i, 0)),
            pl.BlockSpec((BR, CC), lambda i: (i, 0)),
        ),
        out_shape=(jax.ShapeDtypeStruct((2, NP, CH), jnp.float32),
                   jax.ShapeDtypeStruct((NN, CC), jnp.float32),
                   jax.ShapeDtypeStruct((NN, CC), jnp.float32)),
    )(tbl4, tbl4, acc0, acc1, deg, wsage, wgcn0)


def _tc_fin_body(acc2, s2s, s3, sg2, deg, wg1, wg2, out):
    inv = 1.0 / (deg[:, 0:1] + 1.0)
    s2 = _unsplit_blk(s2s)
    a = _unsplit_blk(acc2) * inv
    dot = lambda x, w: jnp.dot(x, w, preferred_element_type=jnp.float32)
    s4 = jnp.maximum(dot(a, wg1[...]), 0.0) + sg2[...]
    s5 = s3[...] + jnp.maximum(dot(a, wg2[...]), 0.0)
    out[...] = jnp.concatenate([s2, s3[...], s4, s5], axis=1)


def _tc_fin(acc2, s2s, s3, sg2, deg, wg1, wg2):
    full = lambda *shape: pl.BlockSpec(shape, lambda i: (0,) * len(shape))
    return pl.pallas_call(
        _tc_fin_body,
        grid=(NB,),
        in_specs=[
            pl.BlockSpec((2, BR, CH), lambda i: (0, i, 0)),
            pl.BlockSpec((2, BR, CH), lambda i: (0, i, 0)),
            pl.BlockSpec((BR, CC), lambda i: (i, 0)),
            pl.BlockSpec((BR, CC), lambda i: (i, 0)),
            pl.BlockSpec((BR, DEGW), lambda i: (i, 0)),
            full(CC, CC),
            full(CC, CC),
        ],
        out_specs=pl.BlockSpec((BR, 4 * CC), lambda i: (i, 0)),
        out_shape=jax.ShapeDtypeStruct((NN, 4 * CC), jnp.float32),
    )(acc2, s2s, s3, sg2, deg, wg1, wg2)


@jax.jit
def _run(s0, s1, edge_index, w_pre, bn_gamma, bn_beta, w_sage, w_gcn):
    pad = EP - EE
    src = jnp.concatenate(
        [edge_index[0].astype(jnp.int32), jnp.zeros((pad,), jnp.int32)])
    dst = jnp.concatenate(
        [edge_index[1].astype(jnp.int32), jnp.full((pad,), NN, jnp.int32)])
    src = src.reshape(NTEC, CPT, CHUNK)
    dst = dst.reshape(NTEC, CPT, CHUNK)
    src1 = src + NP

    tbl = _tc_pre(s0, s1, w_pre, bn_gamma, bn_beta)
    acc0, deg = _sc_agg(tbl, src, dst, do_deg=True, row_off=0)
    acc1 = _sc_agg(tbl, src1, dst, do_deg=False, row_off=NP)
    tbl4 = tbl.reshape(2, 2, NP, CH)
    s2s, s3, sg2 = _tc_mid(tbl4, acc0, acc1, deg, w_sage, w_gcn[0])
    acc2 = _sc_agg(s2s, src, dst, do_deg=False, row_off=0)
    return _tc_fin(acc2, s2s, s3, sg2, deg, w_gcn[1], w_gcn[2])


def kernel(s0, s1, edge_index, drop_prob, W_pre, bn_gamma, bn_beta, W_sage, W_gcn):
    del drop_prob
    return _run(s0, s1, edge_index, W_pre, bn_gamma, bn_beta, W_sage, W_gcn)


# speedup vs baseline: 1.0809x; 1.0809x over previous
import functools
import jax
import jax.numpy as jnp
from jax import lax
from jax.experimental import pallas as pl
from jax.experimental.pallas import tpu as pltpu
from jax.experimental.pallas import tpu_sc as plsc

NN = 10000
NP = 10240
EE = 320000
CC = 128
CH = CC // 2
NSC = 2
NTEC = 16
CHUNK = 80
CPT = EE // NTEC // CHUNK
RPT = NP // NTEC
DEGW = 16

_MESH = dict(core_axis_name="c", subcore_axis_name="s", num_cores=NSC,
             num_subcores=NTEC)


def _sc_agg_body(do_deg, row_off, tbl, esrc, edst, *refs):
    if do_deg:
        (acc_out, deg_out, idx_src, idx_dst, rows, ones_v, zer_v,
         acc_sh, deg_sh, sem, dsem) = refs
    else:
        acc_out, idx_src, idx_dst, rows, acc_sh, sem = refs

    c = lax.axis_index("c")
    s = lax.axis_index("s")
    row0 = s * RPT

    pltpu.sync_copy(esrc.at[s], idx_src)
    pltpu.sync_copy(edst.at[s], idx_dst)

    pltpu.sync_copy(tbl.at[c, pl.ds(row_off + row0, RPT)],
                    acc_sh.at[pl.ds(row0, RPT)])

    if do_deg:
        for r in range(CHUNK):
            ones_v[r, :] = jnp.full((16,), 1.0, jnp.float32)

        @pl.when(c == 1)
        def _():
            for r in range(128):
                zer_v[r, :] = jnp.zeros((16,), jnp.float32)
            for b in range(RPT // 128):
                pltpu.sync_copy(zer_v, deg_sh.at[pl.ds(row0 + b * 128, 128)])

    plsc.subcore_barrier()

    gsem = sem

    def start_gather(j, i):
        pltpu.make_async_copy(tbl.at[c].at[idx_src.at[j]], rows[i],
                              gsem[i]).start()

    def finish_chunk(j, i):
        pltpu.make_async_copy(tbl.at[c].at[idx_src.at[j]], rows[i],
                              gsem[i]).wait()
        pltpu.sync_copy(rows[i], acc_sh.at[idx_dst.at[j]], add=True)
        if do_deg:
            @pl.when(c == 1)
            def _():
                @pl.when(j > 0)
                def _():
                    pltpu.make_async_copy(ones_v, deg_sh.at[idx_dst.at[0]],
                                          dsem).wait()
                pltpu.async_copy(ones_v, deg_sh.at[idx_dst.at[j]], dsem,
                                 add=True)

    start_gather(0, 0)

    def body(k, carry):
        j0 = 2 * k
        start_gather(j0 + 1, 1)
        finish_chunk(j0, 0)

        @pl.when(j0 + 2 < CPT)
        def _():
            start_gather(j0 + 2, 0)
        finish_chunk(j0 + 1, 1)
        return carry

    lax.fori_loop(0, CPT // 2, body, 0, unroll=False)
    if do_deg:
        @pl.when(c == 1)
        def _():
            pltpu.make_async_copy(ones_v, deg_sh.at[idx_dst.at[0]],
                                  dsem).wait()

    plsc.subcore_barrier()

    pltpu.sync_copy(acc_sh.at[pl.ds(row0, RPT)],
                    acc_out.at[c, pl.ds(row0, RPT)])

    if do_deg:
        @pl.when(c == 1)
        def _():
            pltpu.sync_copy(deg_sh.at[pl.ds(row0, RPT)],
                            deg_out.at[pl.ds(row0, RPT)])


def _sc_agg(tbl, esrc, edst, do_deg, row_off):
    out_acc = jax.ShapeDtypeStruct((NSC, NP, CH), jnp.float32)
    sems2 = tuple(pltpu.SemaphoreType.DMA for _ in range(2))
    scratch = [
        pltpu.VMEM((CPT, CHUNK), jnp.int32),
        pltpu.VMEM((CPT, CHUNK), jnp.int32),
        tuple(pltpu.VMEM((CHUNK, CH), jnp.float32) for _ in range(2)),
    ]
    if do_deg:
        out_type = (out_acc, jax.ShapeDtypeStruct((NP, DEGW), jnp.float32))
        scratch += [pltpu.VMEM((CHUNK, DEGW), jnp.float32),
                    pltpu.VMEM((128, DEGW), jnp.float32),
                    pltpu.VMEM_SHARED((NP, CH), jnp.float32),
                    pltpu.VMEM_SHARED((NP, DEGW), jnp.float32),
                    sems2, pltpu.SemaphoreType.DMA]
    else:
        out_type = out_acc
        scratch += [pltpu.VMEM_SHARED((NP, CH), jnp.float32),
                    sems2]
    return pl.kernel(
        functools.partial(_sc_agg_body, do_deg, row_off),
        out_type=out_type,
        mesh=plsc.VectorSubcoreMesh(**_MESH),
        scratch_types=scratch,
        compiler_params=pltpu.CompilerParams(use_tc_tiling_on_sc=False),
    )(tbl, esrc, edst)


def _tc_pre_body(s0, s1, wpre, gamma, beta, tbl_out):
    for i, sref in enumerate((s0, s1)):
        h = jnp.dot(sref[...].astype(jnp.bfloat16), wpre[i].astype(jnp.bfloat16),
                    preferred_element_type=jnp.float32)
        mu = jnp.mean(h, axis=0, keepdims=True)
        var = jnp.mean(jnp.square(h - mu), axis=0, keepdims=True)
        hn = (h - mu) * lax.rsqrt(var + 1e-5) * gamma[i][None, :] + beta[i][None, :]
        p = jnp.maximum(hn, 0.0)
        for half in range(2):
            tbl_out[half, i * NP:i * NP + NN, :] = p[:, half * CH:(half + 1) * CH]


def _tc_pre(s0, s1, wpre, gamma, beta):
    return pl.pallas_call(
        _tc_pre_body,
        out_shape=jax.ShapeDtypeStruct((2, 2 * NP, CH), jnp.float32),
    )(s0, s1, wpre, gamma, beta)


BR = 1000
NB = NN // BR


def _unsplit_blk(a):
    return jnp.concatenate([a[0], a[1]], axis=1)


def _tc_mid_body(tblp0, tblp1, acc0, acc1, deg, wsage, wgcn0, s2o, s3o, sg2o):
    inv = 1.0 / (deg[:, 0:1] + 1.0)
    p0 = jnp.concatenate([tblp0[0, 0], tblp0[1, 0]], axis=1)
    p1 = jnp.concatenate([tblp1[0, 0], tblp1[1, 0]], axis=1)
    a0 = _unsplit_blk(acc0) * inv
    a1 = _unsplit_blk(acc1) * inv
    dot = lambda a, b: jnp.dot(a.astype(jnp.bfloat16), b.astype(jnp.bfloat16),
                               preferred_element_type=jnp.float32)
    r = lambda x: jnp.maximum(x, 0.0)
    s2 = r(dot(p0, wsage[0, 0]) + dot(a0, wsage[0, 1])) + r(dot(a1, wgcn0[...]))
    for half in range(2):
        s2o[half] = s2[:, half * CH:(half + 1) * CH]
    s3o[...] = r(dot(p1, wsage[1, 0]) + dot(a1, wsage[1, 1])) + p0
    sg2o[...] = r(dot(p1, wsage[2, 0]) + dot(a1, wsage[2, 1]))


def _tc_mid(tbl4, acc0, acc1, deg, wsage, wgcn0):
    full = lambda *shape: pl.BlockSpec(shape, lambda i: (0,) * len(shape))
    return pl.pallas_call(
        _tc_mid_body,
        grid=(NB,),
        in_specs=[
            pl.BlockSpec((2, 1, BR, CH), lambda i: (0, 0, i, 0)),
            pl.BlockSpec((2, 1, BR, CH), lambda i: (0, 1, i, 0)),
            pl.BlockSpec((2, BR, CH), lambda i: (0, i, 0)),
            pl.BlockSpec((2, BR, CH), lambda i: (0, i, 0)),
            pl.BlockSpec((BR, DEGW), lambda i: (i, 0)),
            full(3, 2, CC, CC),
            full(CC, CC),
        ],
        out_specs=(
            pl.BlockSpec((2, BR, CH), lambda i: (0, i, 0)),
            pl.BlockSpec((BR, CC), lambda i: (i, 0)),
            pl.BlockSpec((BR, CC), lambda i: (i, 0)),
        ),
        out_shape=(jax.ShapeDtypeStruct((2, NP, CH), jnp.float32),
                   jax.ShapeDtypeStruct((NN, CC), jnp.float32),
                   jax.ShapeDtypeStruct((NN, CC), jnp.float32)),
    )(tbl4, tbl4, acc0, acc1, deg, wsage, wgcn0)


def _tc_fin_body(acc2, s2s, s3, sg2, deg, wg1, wg2, out):
    inv = 1.0 / (deg[:, 0:1] + 1.0)
    s2 = _unsplit_blk(s2s)
    a = _unsplit_blk(acc2) * inv
    dot = lambda x, w: jnp.dot(x.astype(jnp.bfloat16), w.astype(jnp.bfloat16),
                               preferred_element_type=jnp.float32)
    s4 = jnp.maximum(dot(a, wg1[...]), 0.0) + sg2[...]
    s5 = s3[...] + jnp.maximum(dot(a, wg2[...]), 0.0)
    out[...] = jnp.concatenate([s2, s3[...], s4, s5], axis=1)


def _tc_fin(acc2, s2s, s3, sg2, deg, wg1, wg2):
    full = lambda *shape: pl.BlockSpec(shape, lambda i: (0,) * len(shape))
    return pl.pallas_call(
        _tc_fin_body,
        grid=(NB,),
        in_specs=[
            pl.BlockSpec((2, BR, CH), lambda i: (0, i, 0)),
            pl.BlockSpec((2, BR, CH), lambda i: (0, i, 0)),
            pl.BlockSpec((BR, CC), lambda i: (i, 0)),
            pl.BlockSpec((BR, CC), lambda i: (i, 0)),
            pl.BlockSpec((BR, DEGW), lambda i: (i, 0)),
            full(CC, CC),
            full(CC, CC),
        ],
        out_specs=pl.BlockSpec((BR, 4 * CC), lambda i: (i, 0)),
        out_shape=jax.ShapeDtypeStruct((NN, 4 * CC), jnp.float32),
    )(acc2, s2s, s3, sg2, deg, wg1, wg2)


@jax.jit
def _run(s0, s1, edge_index, w_pre, bn_gamma, bn_beta, w_sage, w_gcn):
    src = edge_index[0].astype(jnp.int32).reshape(NTEC, CPT, CHUNK)
    dst = edge_index[1].astype(jnp.int32).reshape(NTEC, CPT, CHUNK)
    src1 = src + NP

    tbl = _tc_pre(s0, s1, w_pre, bn_gamma, bn_beta)
    acc0, deg = _sc_agg(tbl, src, dst, do_deg=True, row_off=0)
    acc1 = _sc_agg(tbl, src1, dst, do_deg=False, row_off=NP)
    tbl4 = tbl.reshape(2, 2, NP, CH)
    s2s, s3, sg2 = _tc_mid(tbl4, acc0, acc1, deg, w_sage, w_gcn[0])
    acc2 = _sc_agg(s2s, src, dst, do_deg=False, row_off=0)
    return _tc_fin(acc2, s2s, s3, sg2, deg, w_gcn[1], w_gcn[2])


def kernel(s0, s1, edge_index, drop_prob, W_pre, bn_gamma, bn_beta, W_sage, W_gcn):
    del drop_prob
    return _run(s0, s1, edge_index, W_pre, bn_gamma, bn_beta, W_sage, W_gcn)
